# baseline (speedup 1.0000x reference)
import numpy as np
import jax
import jax.numpy as jnp
from jax import lax
from jax.experimental import pallas as pl
from jax.experimental.pallas import tpu as pltpu
from jax.experimental.pallas import tpu_sc as plsc

NUM_USER = 100000
NUM_ITEM = 100000
DIM_E = 64
DIM_FEAT = 128
NUM_NEG = 127
BATCH = 1024
K = 1 + NUM_NEG
N = BATCH * K
TEMP_VALUE = 2.0
NUM_SAMPLE = 0.5
CONTRASTIVE = 0.5


def _np_threefry2x32(k0, k1, x0, x1):
    m = np.uint64(0xFFFFFFFF)
    x0 = x0.astype(np.uint64)
    x1 = x1.astype(np.uint64)
    ks = [np.uint64(k0), np.uint64(k1),
          (np.uint64(k0) ^ np.uint64(k1) ^ np.uint64(0x1BD11BDA)) & m]
    rots = [(13, 15, 26, 6), (17, 29, 16, 24)]
    x0 = (x0 + ks[0]) & m
    x1 = (x1 + ks[1]) & m
    for r in range(5):
        for d in rots[r % 2]:
            x0 = (x0 + x1) & m
            x1 = ((x1 << np.uint64(d)) | (x1 >> np.uint64(32 - d))) & m
            x1 = x1 ^ x0
        x0 = (x0 + ks[(r + 1) % 3]) & m
        x1 = (x1 + ks[(r + 2) % 3] + np.uint64(r + 1)) & m
    return x0.astype(np.uint32), x1.astype(np.uint32)


def _np_bits32(k0, k1, n):
    a, b = _np_threefry2x32(k0, k1, np.zeros(n, np.uint32),
                            np.arange(n, dtype=np.uint32))
    return a ^ b


def _np_rand_index():
    sk = _np_threefry2x32(np.uint32(0), np.uint32(42),
                          np.zeros(2, np.uint32), np.arange(2, dtype=np.uint32))
    hi = _np_bits32(sk[0][0], sk[1][0], int(N * NUM_SAMPLE)).astype(np.uint64)
    lo = _np_bits32(sk[0][1], sk[1][1], int(N * NUM_SAMPLE)).astype(np.uint64)
    span = np.uint64(N)
    mult = (np.uint64(2 ** 16) % span)
    mult = (mult * mult) % span
    return (((hi % span) * mult + (lo % span)) % span).astype(np.int64)


_MASK_NP = np.zeros((N,), np.float32)
_MASK_NP[_np_rand_index()] = 1.0

_MLP_BLK = 2000


def _mlp_body(v_ref, w1_ref, b1_ref, w2_ref, b2_ref, out_ref):
    x = v_ref[...]
    nrm = jnp.sqrt(jnp.sum(x * x, axis=1, keepdims=True))
    x = x / jnp.maximum(nrm, 1e-12)
    h = jnp.dot(x, w1_ref[...], preferred_element_type=jnp.float32) + b1_ref[...]
    h = jnp.where(h >= 0, h, 0.01 * h)
    y = jnp.dot(h, w2_ref[...], preferred_element_type=jnp.float32) + b2_ref[...]
    out_ref[:, 0:DIM_E] = y
    out_ref[:, DIM_E:128] = jnp.zeros((_MLP_BLK, 128 - DIM_E), jnp.float32)


def _mlp(v_feat, W1, b1, W2, b2):
    grid = NUM_ITEM // _MLP_BLK
    return pl.pallas_call(
        _mlp_body,
        grid=(grid,),
        in_specs=[
            pl.BlockSpec((_MLP_BLK, DIM_FEAT), lambda i: (i, 0)),
            pl.BlockSpec((DIM_FEAT, 256), lambda i: (0, 0)),
            pl.BlockSpec((1, 256), lambda i: (0, 0)),
            pl.BlockSpec((256, DIM_E), lambda i: (0, 0)),
            pl.BlockSpec((1, DIM_E), lambda i: (0, 0)),
        ],
        out_specs=pl.BlockSpec((_MLP_BLK, 128), lambda i: (i, 0)),
        out_shape=jax.ShapeDtypeStruct((NUM_ITEM, 128), jnp.float32),
    )(v_feat, W1, b1.reshape(1, 256), W2, b2.reshape(1, DIM_E))


_PAD_BLK = 4000


def _pad_body(x_ref, out_ref):
    out_ref[:, 0:DIM_E] = x_ref[...]
    out_ref[:, DIM_E:128] = jnp.zeros((_PAD_BLK, 128 - DIM_E), jnp.float32)


def _pad128(x):
    rows = x.shape[0]
    grid = rows // _PAD_BLK
    return pl.pallas_call(
        _pad_body,
        grid=(grid,),
        in_specs=[pl.BlockSpec((_PAD_BLK, DIM_E), lambda i: (i, 0))],
        out_specs=pl.BlockSpec((_PAD_BLK, 128), lambda i: (i, 0)),
        out_shape=jax.ShapeDtypeStruct((rows, 128), jnp.float32),
    )(x)


_NW = 32
_PW = N // _NW
_CH = 64
_SUB = 128


def _sc_body(emb, feat, uidx_h, iidx_h, fidx_h,
             uu_o, ee_o, ff_o, fp_o, ue_o, uf_o,
             uidx_v, iidx_v, fidx_v,
             u_r0, e_r0, f_r0, u_r1, e_r1, f_r1,
             pa0, pa1, pa2, pa3, pa4, pa5,
             pb0, pb1, pb2, pb3, pb4, pb5,
             sg0, sg1, so0, so1):
    wid = lax.axis_index("s") * 2 + lax.axis_index("c")
    base = wid * _PW
    pltpu.sync_copy(uidx_h.at[pl.ds(base, _PW)], uidx_v)
    pltpu.sync_copy(iidx_h.at[pl.ds(base, _PW)], iidx_v)
    pltpu.sync_copy(fidx_h.at[pl.ds(base, _PW)], fidx_v)

    outs = (uu_o, ee_o, ff_o, fp_o, ue_o, uf_o)
    bufs0 = (u_r0, e_r0, f_r0)
    bufs1 = (u_r1, e_r1, f_r1)
    parts0 = (pa0, pa1, pa2, pa3, pa4, pa5)
    parts1 = (pb0, pb1, pb2, pb3, pb4, pb5)
    nchunks = _PW // _CH
    half = nchunks // 2

    def issue_gathers(ci, bufs, sem):
        sl = pl.ds(ci * _CH, _CH)
        pltpu.async_copy(emb.at[uidx_v.at[sl]], bufs[0], sem)
        pltpu.async_copy(emb.at[iidx_v.at[sl]], bufs[1], sem)
        pltpu.async_copy(feat.at[fidx_v.at[sl]], bufs[2], sem)

    def wait_gathers(bufs, sem):
        for b in bufs:
            pltpu.make_async_copy(emb.at[pl.ds(0, _CH)], b, sem).wait()

    def wait_parts(parts, sem):
        for k, p in enumerate(parts):
            pltpu.make_async_copy(outs[k].at[pl.ds(0, _CH * 16)], p, sem).wait()

    def issue_parts(ci, parts, sem):
        dst = pl.ds((base + ci * _CH) * 16, _CH * 16)
        for k, p in enumerate(parts):
            pltpu.async_copy(p, outs[k].at[dst], sem)

    def compute(bufs, parts, p):
        u_rows, e_rows, f_rows = bufs
        p0, p1, p2, p3 = p

        def row(r, c3):
            u0 = u_rows[r, pl.ds(0, 16)]
            u1 = u_rows[r, pl.ds(16, 16)]
            u2 = u_rows[r, pl.ds(32, 16)]
            u3 = u_rows[r, pl.ds(48, 16)]
            e0 = e_rows[r, pl.ds(0, 16)]
            e1 = e_rows[r, pl.ds(16, 16)]
            e2 = e_rows[r, pl.ds(32, 16)]
            e3 = e_rows[r, pl.ds(48, 16)]
            f0 = f_rows[r, pl.ds(0, 16)]
            f1 = f_rows[r, pl.ds(16, 16)]
            f2 = f_rows[r, pl.ds(32, 16)]
            f3 = f_rows[r, pl.ds(48, 16)]
            sl = pl.ds(r * 16, 16)
            parts[0][sl] = u0 * u0 + u1 * u1 + u2 * u2 + u3 * u3
            parts[1][sl] = e0 * e0 + e1 * e1 + e2 * e2 + e3 * e3
            parts[2][sl] = f0 * f0 + f1 * f1 + f2 * f2 + f3 * f3
            parts[3][sl] = f0 * p0 + f1 * p1 + f2 * p2 + f3 * p3
            parts[4][sl] = u0 * e0 + u1 * e1 + u2 * e2 + u3 * e3
            parts[5][sl] = u0 * f0 + u1 * f1 + u2 * f2 + u3 * f3
            return c3

        lax.fori_loop(0, _CH, row, 0, unroll=4)

    issue_gathers(0, bufs0, sg0)
    issue_gathers(1, bufs1, sg1)

    def step(ii, carry):
        ci0 = 2 * ii
        ci1 = 2 * ii + 1

        wait_gathers(bufs0, sg0)
        e_rows0 = bufs0[1]
        p = (e_rows0[0, pl.ds(0, 16)], e_rows0[0, pl.ds(16, 16)],
             e_rows0[0, pl.ds(32, 16)], e_rows0[0, pl.ds(48, 16)])

        @pl.when(ii > 0)
        def _():
            wait_parts(parts0, so0)
        compute(bufs0, parts0, p)
        issue_parts(ci0, parts0, so0)

        wait_gathers(bufs1, sg1)

        @pl.when(ii > 0)
        def _():
            wait_parts(parts1, so1)
        compute(bufs1, parts1, p)
        issue_parts(ci1, parts1, so1)

        @pl.when(ii < half - 1)
        def _():
            issue_gathers(ci0 + 2, bufs0, sg0)
            issue_gathers(ci1 + 2, bufs1, sg1)
        return carry

    lax.fori_loop(0, half, step, 0)
    wait_parts(parts0, so0)
    wait_parts(parts1, so1)


def _sc_dots(id_embedding, feature, uidx, iidx, fidx):
    mesh = plsc.VectorSubcoreMesh(core_axis_name="c", subcore_axis_name="s")
    vec = jax.ShapeDtypeStruct((N * 16,), jnp.float32)
    ivec = pltpu.VMEM((_PW,), jnp.int32)
    rows = pltpu.VMEM((_CH, 128), jnp.float32)
    part = pltpu.VMEM((_CH * 16,), jnp.float32)
    return pl.kernel(
        _sc_body,
        out_type=(vec, vec, vec, vec, vec, vec),
        mesh=mesh,
        scratch_types=[
            ivec, ivec, ivec,
            rows, rows, rows, rows, rows, rows,
            part, part, part, part, part, part,
            part, part, part, part, part, part,
            pltpu.SemaphoreType.DMA, pltpu.SemaphoreType.DMA,
            pltpu.SemaphoreType.DMA, pltpu.SemaphoreType.DMA,
        ],
        compiler_params=pltpu.CompilerParams(use_tc_tiling_on_sc=False,
                                             needs_layout_passes=False),
    )(id_embedding, feature, uidx, iidx, fidx)


_FGRID = 8
_FQ = (N // 8) // _FGRID
_FG = 128


def _fin_body(uu_ref, ee_ref, ff_ref, fp_ref, ue_ref, uf_ref, mp_ref,
              s1_ref, s2_ref, ru_ref, re_ref):
    step = pl.program_id(0)
    li = lax.broadcasted_iota(jnp.int32, (128, 8), 0) // 16
    ki = lax.broadcasted_iota(jnp.int32, (128, 8), 1)
    sel = (li == ki).astype(jnp.float32)
    red = lambda ref: lax.dot_general(ref[...], sel, (((1,), (0,)), ((), ())),
                                      preferred_element_type=jnp.float32)
    uu = red(uu_ref)
    ee = red(ee_ref)
    ff = red(ff_ref)
    fp = red(fp_ref)
    ue = red(ue_ref)
    uf = red(uf_ref)
    mp = mp_ref[...]
    um = ue + mp * (uf - ue)

    qi = lax.broadcasted_iota(jnp.int32, (_FG, _FQ), 1) // 16
    gi = lax.broadcasted_iota(jnp.int32, (_FG, _FQ), 0)
    gsel = (qi == gi).astype(jnp.float32)
    gsum = lambda x: jnp.sum(lax.dot_general(
        gsel, x, (((1,), (0,)), ((), ())),
        preferred_element_type=jnp.float32), axis=1)

    q0 = (lax.broadcasted_iota(jnp.int32, (_FQ, 8), 0) % 16 == 0)
    k0 = (lax.broadcasted_iota(jnp.int32, (_FQ, 8), 1) == 0)
    pm = (q0 & k0).astype(jnp.float32)

    pp = gsum(ee * pm)
    ppr = lax.dot_general(gsel, pp.reshape(_FG, 1), (((0,), (0,)), ((), ())),
                          preferred_element_type=jnp.float32)

    nf = jnp.maximum(jnp.sqrt(ff), 1e-12)
    npp = jnp.maximum(jnp.sqrt(ppr), 1e-12)
    d1 = fp / (nf * npp)
    e1 = jnp.exp(d1 * (1.0 / TEMP_VALUE))
    e2 = jnp.exp(um * (1.0 / TEMP_VALUE))

    tot1 = gsum(e1)
    tot2 = gsum(e2)
    pos1 = gsum(e1 * pm)
    pos2 = gsum(e2 * pm)
    s1_ref[step, :] = -jnp.log(pos1 / tot1) * CONTRASTIVE
    s2_ref[step, :] = -jnp.log(pos2 / tot2) * (1.0 - CONTRASTIVE)

    su = jnp.sum(jnp.sqrt(uu)).reshape(1, 1)
    se = jnp.sum(jnp.sqrt(ee)).reshape(1, 1)

    @pl.when(step == 0)
    def _():
        ru_ref[...] = jnp.zeros_like(ru_ref)
        re_ref[...] = jnp.zeros_like(re_ref)

    ru_ref[...] += su
    re_ref[...] += se


def _finalize(uu, ee, ff, fp, ue, uf, maskp):
    spec = pl.BlockSpec((_FQ, 128), lambda i: (i, 0))
    mspec = pl.BlockSpec((_FQ, 8), lambda i: (i, 0))
    return pl.pallas_call(
        _fin_body,
        grid=(_FGRID,),
        in_specs=[spec, spec, spec, spec, spec, spec, mspec],
        out_specs=[
            pl.BlockSpec((_FGRID, _FG), lambda i: (0, 0)),
            pl.BlockSpec((_FGRID, _FG), lambda i: (0, 0)),
            pl.BlockSpec((1, 1), lambda i: (0, 0)),
            pl.BlockSpec((1, 1), lambda i: (0, 0)),
        ],
        out_shape=(
            jax.ShapeDtypeStruct((_FGRID, _FG), jnp.float32),
            jax.ShapeDtypeStruct((_FGRID, _FG), jnp.float32),
            jax.ShapeDtypeStruct((1, 1), jnp.float32),
            jax.ShapeDtypeStruct((1, 1), jnp.float32),
        ),
    )(uu, ee, ff, fp, ue, uf, maskp)


def kernel(user_tensor, item_tensor, id_embedding, v_feat, W1, b1, W2, b2):
    feature = _mlp(v_feat, W1, b1, W2, b2)

    uidx = user_tensor.reshape(-1).astype(jnp.int32)
    iidx = item_tensor.reshape(-1).astype(jnp.int32)
    fidx = iidx - NUM_USER

    emb_p = _pad128(id_embedding)
    uu, ee, ff, fp, ue, uf = _sc_dots(emb_p, feature, uidx, iidx, fidx)

    maskp = jnp.asarray(_MASK_NP.reshape(N // 8, 8))
    pk = (N // 8, 128)
    s1m, s2m, ru, re = _finalize(uu.reshape(pk), ee.reshape(pk), ff.reshape(pk),
                                 fp.reshape(pk), ue.reshape(pk), uf.reshape(pk),
                                 maskp)
    reg = (ru[0, 0] + re[0, 0]) / (2.0 * N)
    return (s1m.reshape(-1), s2m.reshape(-1), reg)


# speedup vs baseline: 1.0987x; 1.0987x over previous
import numpy as np
import jax
import jax.numpy as jnp
from jax import lax
from jax.experimental import pallas as pl
from jax.experimental.pallas import tpu as pltpu
from jax.experimental.pallas import tpu_sc as plsc

NUM_USER = 100000
NUM_ITEM = 100000
DIM_E = 64
DIM_FEAT = 128
NUM_NEG = 127
BATCH = 1024
K = 1 + NUM_NEG
N = BATCH * K
TEMP_VALUE = 2.0
NUM_SAMPLE = 0.5
CONTRASTIVE = 0.5


def _np_threefry2x32(k0, k1, x0, x1):
    m = np.uint64(0xFFFFFFFF)
    x0 = x0.astype(np.uint64)
    x1 = x1.astype(np.uint64)
    ks = [np.uint64(k0), np.uint64(k1),
          (np.uint64(k0) ^ np.uint64(k1) ^ np.uint64(0x1BD11BDA)) & m]
    rots = [(13, 15, 26, 6), (17, 29, 16, 24)]
    x0 = (x0 + ks[0]) & m
    x1 = (x1 + ks[1]) & m
    for r in range(5):
        for d in rots[r % 2]:
            x0 = (x0 + x1) & m
            x1 = ((x1 << np.uint64(d)) | (x1 >> np.uint64(32 - d))) & m
            x1 = x1 ^ x0
        x0 = (x0 + ks[(r + 1) % 3]) & m
        x1 = (x1 + ks[(r + 2) % 3] + np.uint64(r + 1)) & m
    return x0.astype(np.uint32), x1.astype(np.uint32)


def _np_bits32(k0, k1, n):
    a, b = _np_threefry2x32(k0, k1, np.zeros(n, np.uint32),
                            np.arange(n, dtype=np.uint32))
    return a ^ b


def _np_rand_index():
    sk = _np_threefry2x32(np.uint32(0), np.uint32(42),
                          np.zeros(2, np.uint32), np.arange(2, dtype=np.uint32))
    hi = _np_bits32(sk[0][0], sk[1][0], int(N * NUM_SAMPLE)).astype(np.uint64)
    lo = _np_bits32(sk[0][1], sk[1][1], int(N * NUM_SAMPLE)).astype(np.uint64)
    span = np.uint64(N)
    mult = (np.uint64(2 ** 16) % span)
    mult = (mult * mult) % span
    return (((hi % span) * mult + (lo % span)) % span).astype(np.int64)


_MASK_NP = np.zeros((N,), np.float32)
_MASK_NP[_np_rand_index()] = 1.0

_MLP_BLK = 2000


def _mlp_body(v_ref, w1_ref, b1_ref, w2_ref, b2_ref, out_ref):
    x = v_ref[...]
    nrm = jnp.sqrt(jnp.sum(x * x, axis=1, keepdims=True))
    x = x / jnp.maximum(nrm, 1e-12)
    h = jnp.dot(x, w1_ref[...], preferred_element_type=jnp.float32) + b1_ref[...]
    h = jnp.where(h >= 0, h, 0.01 * h)
    y = jnp.dot(h, w2_ref[...], preferred_element_type=jnp.float32) + b2_ref[...]
    out_ref[...] = y


def _mlp(v_feat, W1, b1, W2, b2):
    grid = NUM_ITEM // _MLP_BLK
    return pl.pallas_call(
        _mlp_body,
        grid=(grid,),
        in_specs=[
            pl.BlockSpec((_MLP_BLK, DIM_FEAT), lambda i: (i, 0)),
            pl.BlockSpec((DIM_FEAT, 256), lambda i: (0, 0)),
            pl.BlockSpec((1, 256), lambda i: (0, 0)),
            pl.BlockSpec((256, DIM_E), lambda i: (0, 0)),
            pl.BlockSpec((1, DIM_E), lambda i: (0, 0)),
        ],
        out_specs=pl.BlockSpec((_MLP_BLK, DIM_E), lambda i: (i, 0)),
        out_shape=jax.ShapeDtypeStruct((NUM_ITEM, DIM_E), jnp.float32),
    )(v_feat, W1, b1.reshape(1, 256), W2, b2.reshape(1, DIM_E))


_PAD_BLK = 4000


def _pad_body(x_ref, out_ref):
    out_ref[:, 0:DIM_E] = x_ref[...]
    out_ref[:, DIM_E:128] = jnp.zeros((_PAD_BLK, 128 - DIM_E), jnp.float32)


def _pad128(x):
    rows = x.shape[0]
    grid = rows // _PAD_BLK
    return pl.pallas_call(
        _pad_body,
        grid=(grid,),
        in_specs=[pl.BlockSpec((_PAD_BLK, DIM_E), lambda i: (i, 0))],
        out_specs=pl.BlockSpec((_PAD_BLK, 128), lambda i: (i, 0)),
        out_shape=jax.ShapeDtypeStruct((rows, 128), jnp.float32),
    )(x)


_NW = 32
_PW = N // _NW
_CH = 128
_SUB = 128


def _sc_body(emb, feat, uidx_h, iidx_h, fidx_h,
             uu_o, ee_o, ff_o, fp_o, ue_o, uf_o,
             uidx_v, iidx_v, fidx_v,
             u_r0, e_r0, f_r0, u_r1, e_r1, f_r1,
             pa0, pa1, pa2, pa3, pa4, pa5,
             pb0, pb1, pb2, pb3, pb4, pb5,
             sg0, sg1, so0, so1):
    wid = lax.axis_index("s") * 2 + lax.axis_index("c")
    base = wid * _PW
    pltpu.sync_copy(uidx_h.at[pl.ds(base, _PW)], uidx_v)
    pltpu.sync_copy(iidx_h.at[pl.ds(base, _PW)], iidx_v)
    pltpu.sync_copy(fidx_h.at[pl.ds(base, _PW)], fidx_v)

    outs = (uu_o, ee_o, ff_o, fp_o, ue_o, uf_o)
    bufs0 = (u_r0, e_r0, f_r0)
    bufs1 = (u_r1, e_r1, f_r1)
    parts0 = (pa0, pa1, pa2, pa3, pa4, pa5)
    parts1 = (pb0, pb1, pb2, pb3, pb4, pb5)
    nchunks = _PW // _CH
    half = nchunks // 2

    def issue_gathers(ci, bufs, sem):
        sl = pl.ds(ci * _CH, _CH)
        pltpu.async_copy(emb.at[uidx_v.at[sl]], bufs[0], sem)
        pltpu.async_copy(emb.at[iidx_v.at[sl]], bufs[1], sem)
        pltpu.async_copy(feat.at[fidx_v.at[sl]], bufs[2], sem)

    def wait_gathers(bufs, sem):
        pltpu.make_async_copy(emb.at[pl.ds(0, _CH)], bufs[0], sem).wait()
        pltpu.make_async_copy(emb.at[pl.ds(0, _CH)], bufs[1], sem).wait()
        pltpu.make_async_copy(feat.at[pl.ds(0, _CH)], bufs[2], sem).wait()

    def wait_parts(parts, sem):
        for k, p in enumerate(parts):
            pltpu.make_async_copy(outs[k].at[pl.ds(0, _CH * 16)], p, sem).wait()

    def issue_parts(ci, parts, sem):
        dst = pl.ds((base + ci * _CH) * 16, _CH * 16)
        for k, p in enumerate(parts):
            pltpu.async_copy(p, outs[k].at[dst], sem)

    def compute(bufs, parts):
        u_rows, e_rows, f_rows = bufs
        p0 = e_rows[0, pl.ds(0, 16)]
        p1 = e_rows[0, pl.ds(16, 16)]
        p2 = e_rows[0, pl.ds(32, 16)]
        p3 = e_rows[0, pl.ds(48, 16)]

        def row(r, c3):
            u0 = u_rows[r, pl.ds(0, 16)]
            u1 = u_rows[r, pl.ds(16, 16)]
            u2 = u_rows[r, pl.ds(32, 16)]
            u3 = u_rows[r, pl.ds(48, 16)]
            e0 = e_rows[r, pl.ds(0, 16)]
            e1 = e_rows[r, pl.ds(16, 16)]
            e2 = e_rows[r, pl.ds(32, 16)]
            e3 = e_rows[r, pl.ds(48, 16)]
            f0 = f_rows[r, pl.ds(0, 16)]
            f1 = f_rows[r, pl.ds(16, 16)]
            f2 = f_rows[r, pl.ds(32, 16)]
            f3 = f_rows[r, pl.ds(48, 16)]
            sl = pl.ds(r * 16, 16)
            parts[0][sl] = u0 * u0 + u1 * u1 + u2 * u2 + u3 * u3
            parts[1][sl] = e0 * e0 + e1 * e1 + e2 * e2 + e3 * e3
            parts[2][sl] = f0 * f0 + f1 * f1 + f2 * f2 + f3 * f3
            parts[3][sl] = f0 * p0 + f1 * p1 + f2 * p2 + f3 * p3
            parts[4][sl] = u0 * e0 + u1 * e1 + u2 * e2 + u3 * e3
            parts[5][sl] = u0 * f0 + u1 * f1 + u2 * f2 + u3 * f3
            return c3

        lax.fori_loop(0, _CH, row, 0, unroll=4)

    issue_gathers(0, bufs0, sg0)
    issue_gathers(1, bufs1, sg1)

    def step(ii, carry):
        ci0 = 2 * ii
        ci1 = 2 * ii + 1

        @pl.when(ii > 0)
        def _():
            wait_parts(parts0, so0)
        wait_gathers(bufs0, sg0)
        compute(bufs0, parts0)
        issue_parts(ci0, parts0, so0)

        @pl.when(ii < half - 1)
        def _():
            issue_gathers(ci0 + 2, bufs0, sg0)

        @pl.when(ii > 0)
        def _():
            wait_parts(parts1, so1)
        wait_gathers(bufs1, sg1)
        compute(bufs1, parts1)
        issue_parts(ci1, parts1, so1)

        @pl.when(ii < half - 1)
        def _():
            issue_gathers(ci1 + 2, bufs1, sg1)
        return carry

    lax.fori_loop(0, half, step, 0)
    wait_parts(parts0, so0)
    wait_parts(parts1, so1)


def _sc_dots(id_embedding, feature, uidx, iidx, fidx):
    mesh = plsc.VectorSubcoreMesh(core_axis_name="c", subcore_axis_name="s")
    vec = jax.ShapeDtypeStruct((N * 16,), jnp.float32)
    ivec = pltpu.VMEM((_PW,), jnp.int32)
    rows_ue = pltpu.VMEM((_CH, 128), jnp.float32)
    rows_f = pltpu.VMEM((_CH, DIM_E), jnp.float32)
    part = pltpu.VMEM((_CH * 16,), jnp.float32)
    return pl.kernel(
        _sc_body,
        out_type=(vec, vec, vec, vec, vec, vec),
        mesh=mesh,
        scratch_types=[
            ivec, ivec, ivec,
            rows_ue, rows_ue, rows_f, rows_ue, rows_ue, rows_f,
            part, part, part, part, part, part,
            part, part, part, part, part, part,
            pltpu.SemaphoreType.DMA, pltpu.SemaphoreType.DMA,
            pltpu.SemaphoreType.DMA, pltpu.SemaphoreType.DMA,
        ],
        compiler_params=pltpu.CompilerParams(use_tc_tiling_on_sc=False,
                                             needs_layout_passes=False),
    )(id_embedding, feature, uidx, iidx, fidx)


_FGRID = 8
_FQ = (N // 8) // _FGRID
_FG = 128


def _fin_body(uu_ref, ee_ref, ff_ref, fp_ref, ue_ref, uf_ref, mp_ref,
              s1_ref, s2_ref, ru_ref, re_ref):
    step = pl.program_id(0)
    li = lax.broadcasted_iota(jnp.int32, (128, 8), 0) // 16
    ki = lax.broadcasted_iota(jnp.int32, (128, 8), 1)
    sel = (li == ki).astype(jnp.float32)
    red = lambda ref: lax.dot_general(ref[...], sel, (((1,), (0,)), ((), ())),
                                      preferred_element_type=jnp.float32)
    uu = red(uu_ref)
    ee = red(ee_ref)
    ff = red(ff_ref)
    fp = red(fp_ref)
    ue = red(ue_ref)
    uf = red(uf_ref)
    mp = mp_ref[...]
    um = ue + mp * (uf - ue)

    qi = lax.broadcasted_iota(jnp.int32, (_FG, _FQ), 1) // 16
    gi = lax.broadcasted_iota(jnp.int32, (_FG, _FQ), 0)
    gsel = (qi == gi).astype(jnp.float32)
    gsum = lambda x: jnp.sum(lax.dot_general(
        gsel, x, (((1,), (0,)), ((), ())),
        preferred_element_type=jnp.float32), axis=1)

    q0 = (lax.broadcasted_iota(jnp.int32, (_FQ, 8), 0) % 16 == 0)
    k0 = (lax.broadcasted_iota(jnp.int32, (_FQ, 8), 1) == 0)
    pm = (q0 & k0).astype(jnp.float32)

    pp = gsum(ee * pm)
    ppr = lax.dot_general(gsel, pp.reshape(_FG, 1), (((0,), (0,)), ((), ())),
                          preferred_element_type=jnp.float32)

    nf = jnp.maximum(jnp.sqrt(ff), 1e-12)
    npp = jnp.maximum(jnp.sqrt(ppr), 1e-12)
    d1 = fp / (nf * npp)
    e1 = jnp.exp(d1 * (1.0 / TEMP_VALUE))
    e2 = jnp.exp(um * (1.0 / TEMP_VALUE))

    tot1 = gsum(e1)
    tot2 = gsum(e2)
    pos1 = gsum(e1 * pm)
    pos2 = gsum(e2 * pm)
    s1_ref[step, :] = -jnp.log(pos1 / tot1) * CONTRASTIVE
    s2_ref[step, :] = -jnp.log(pos2 / tot2) * (1.0 - CONTRASTIVE)

    su = jnp.sum(jnp.sqrt(uu)).reshape(1, 1)
    se = jnp.sum(jnp.sqrt(ee)).reshape(1, 1)

    @pl.when(step == 0)
    def _():
        ru_ref[...] = jnp.zeros_like(ru_ref)
        re_ref[...] = jnp.zeros_like(re_ref)

    ru_ref[...] += su
    re_ref[...] += se


def _finalize(uu, ee, ff, fp, ue, uf, maskp):
    spec = pl.BlockSpec((_FQ, 128), lambda i: (i, 0))
    mspec = pl.BlockSpec((_FQ, 8), lambda i: (i, 0))
    return pl.pallas_call(
        _fin_body,
        grid=(_FGRID,),
        in_specs=[spec, spec, spec, spec, spec, spec, mspec],
        out_specs=[
            pl.BlockSpec((_FGRID, _FG), lambda i: (0, 0)),
            pl.BlockSpec((_FGRID, _FG), lambda i: (0, 0)),
            pl.BlockSpec((1, 1), lambda i: (0, 0)),
            pl.BlockSpec((1, 1), lambda i: (0, 0)),
        ],
        out_shape=(
            jax.ShapeDtypeStruct((_FGRID, _FG), jnp.float32),
            jax.ShapeDtypeStruct((_FGRID, _FG), jnp.float32),
            jax.ShapeDtypeStruct((1, 1), jnp.float32),
            jax.ShapeDtypeStruct((1, 1), jnp.float32),
        ),
    )(uu, ee, ff, fp, ue, uf, maskp)


def kernel(user_tensor, item_tensor, id_embedding, v_feat, W1, b1, W2, b2):
    feature = _mlp(v_feat, W1, b1, W2, b2)

    uidx = user_tensor.reshape(-1).astype(jnp.int32)
    iidx = item_tensor.reshape(-1).astype(jnp.int32)
    fidx = iidx - NUM_USER

    emb_p = _pad128(id_embedding)
    uu, ee, ff, fp, ue, uf = _sc_dots(emb_p, feature, uidx, iidx, fidx)

    maskp = jnp.asarray(_MASK_NP.reshape(N // 8, 8))
    pk = (N // 8, 128)
    s1m, s2m, ru, re = _finalize(uu.reshape(pk), ee.reshape(pk), ff.reshape(pk),
                                 fp.reshape(pk), ue.reshape(pk), uf.reshape(pk),
                                 maskp)
    reg = (ru[0, 0] + re[0, 0]) / (2.0 * N)
    return (s1m.reshape(-1), s2m.reshape(-1), reg)


# speedup vs baseline: 1.2768x; 1.1621x over previous
import numpy as np
import jax
import jax.numpy as jnp
from jax import lax
from jax.experimental import pallas as pl
from jax.experimental.pallas import tpu as pltpu
from jax.experimental.pallas import tpu_sc as plsc

NUM_USER = 100000
NUM_ITEM = 100000
DIM_E = 64
DIM_FEAT = 128
NUM_NEG = 127
BATCH = 1024
K = 1 + NUM_NEG
N = BATCH * K
TEMP_VALUE = 2.0
NUM_SAMPLE = 0.5
CONTRASTIVE = 0.5


def _np_threefry2x32(k0, k1, x0, x1):
    m = np.uint64(0xFFFFFFFF)
    x0 = x0.astype(np.uint64)
    x1 = x1.astype(np.uint64)
    ks = [np.uint64(k0), np.uint64(k1),
          (np.uint64(k0) ^ np.uint64(k1) ^ np.uint64(0x1BD11BDA)) & m]
    rots = [(13, 15, 26, 6), (17, 29, 16, 24)]
    x0 = (x0 + ks[0]) & m
    x1 = (x1 + ks[1]) & m
    for r in range(5):
        for d in rots[r % 2]:
            x0 = (x0 + x1) & m
            x1 = ((x1 << np.uint64(d)) | (x1 >> np.uint64(32 - d))) & m
            x1 = x1 ^ x0
        x0 = (x0 + ks[(r + 1) % 3]) & m
        x1 = (x1 + ks[(r + 2) % 3] + np.uint64(r + 1)) & m
    return x0.astype(np.uint32), x1.astype(np.uint32)


def _np_bits32(k0, k1, n):
    a, b = _np_threefry2x32(k0, k1, np.zeros(n, np.uint32),
                            np.arange(n, dtype=np.uint32))
    return a ^ b


def _np_rand_index():
    sk = _np_threefry2x32(np.uint32(0), np.uint32(42),
                          np.zeros(2, np.uint32), np.arange(2, dtype=np.uint32))
    hi = _np_bits32(sk[0][0], sk[1][0], int(N * NUM_SAMPLE)).astype(np.uint64)
    lo = _np_bits32(sk[0][1], sk[1][1], int(N * NUM_SAMPLE)).astype(np.uint64)
    span = np.uint64(N)
    mult = (np.uint64(2 ** 16) % span)
    mult = (mult * mult) % span
    return (((hi % span) * mult + (lo % span)) % span).astype(np.int64)


_MASK_NP = np.zeros((N,), np.float32)
_MASK_NP[_np_rand_index()] = 1.0

_MLP_BLK = 2000


def _mlp_body(v_ref, w1_ref, b1_ref, w2_ref, b2_ref, out_ref):
    x = v_ref[...]
    nrm = jnp.sqrt(jnp.sum(x * x, axis=1, keepdims=True))
    x = x / jnp.maximum(nrm, 1e-12)
    h = jnp.dot(x, w1_ref[...], preferred_element_type=jnp.float32) + b1_ref[...]
    h = jnp.where(h >= 0, h, 0.01 * h)
    y = jnp.dot(h, w2_ref[...], preferred_element_type=jnp.float32) + b2_ref[...]
    out_ref[...] = y


def _mlp(v_feat, W1, b1, W2, b2):
    grid = NUM_ITEM // _MLP_BLK
    return pl.pallas_call(
        _mlp_body,
        grid=(grid,),
        in_specs=[
            pl.BlockSpec((_MLP_BLK, DIM_FEAT), lambda i: (i, 0)),
            pl.BlockSpec((DIM_FEAT, 256), lambda i: (0, 0)),
            pl.BlockSpec((1, 256), lambda i: (0, 0)),
            pl.BlockSpec((256, DIM_E), lambda i: (0, 0)),
            pl.BlockSpec((1, DIM_E), lambda i: (0, 0)),
        ],
        out_specs=pl.BlockSpec((_MLP_BLK, DIM_E), lambda i: (i, 0)),
        out_shape=jax.ShapeDtypeStruct((NUM_ITEM, DIM_E), jnp.float32),
    )(v_feat, W1, b1.reshape(1, 256), W2, b2.reshape(1, DIM_E))


_NW = 32
_PW = N // _NW
_CH = 128
_SUB = 128


def _sc_body(emb, feat, uidx_h, iidx_h, fidx_h,
             uu_o, ee_o, ff_o, fp_o, ue_o, uf_o,
             uidx_v, iidx_v, fidx_v,
             u_r0, e_r0, f_r0, u_r1, e_r1, f_r1,
             pa0, pa1, pa2, pa3, pa4, pa5,
             pb0, pb1, pb2, pb3, pb4, pb5,
             sg0, sg1, so0, so1):
    wid = lax.axis_index("s") * 2 + lax.axis_index("c")
    base = wid * _PW
    pltpu.sync_copy(uidx_h.at[pl.ds(base, _PW)], uidx_v)
    pltpu.sync_copy(iidx_h.at[pl.ds(base, _PW)], iidx_v)
    pltpu.sync_copy(fidx_h.at[pl.ds(base, _PW)], fidx_v)

    outs = (uu_o, ee_o, ff_o, fp_o, ue_o, uf_o)
    bufs0 = (u_r0, e_r0, f_r0)
    bufs1 = (u_r1, e_r1, f_r1)
    parts0 = (pa0, pa1, pa2, pa3, pa4, pa5)
    parts1 = (pb0, pb1, pb2, pb3, pb4, pb5)
    nchunks = _PW // _CH
    half = nchunks // 2

    def issue_gathers(ci, bufs, sem):
        sl = pl.ds(ci * _CH, _CH)
        pltpu.async_copy(emb.at[uidx_v.at[sl]], bufs[0], sem)
        pltpu.async_copy(emb.at[iidx_v.at[sl]], bufs[1], sem)
        pltpu.async_copy(feat.at[fidx_v.at[sl]], bufs[2], sem)

    def wait_gathers(bufs, sem):
        pltpu.make_async_copy(emb.at[pl.ds(0, _CH)], bufs[0], sem).wait()
        pltpu.make_async_copy(emb.at[pl.ds(0, _CH)], bufs[1], sem).wait()
        pltpu.make_async_copy(feat.at[pl.ds(0, _CH)], bufs[2], sem).wait()

    def wait_parts(parts, sem):
        for k, p in enumerate(parts):
            pltpu.make_async_copy(outs[k].at[pl.ds(0, _CH * 16)], p, sem).wait()

    def issue_parts(ci, parts, sem):
        dst = pl.ds((base + ci * _CH) * 16, _CH * 16)
        for k, p in enumerate(parts):
            pltpu.async_copy(p, outs[k].at[dst], sem)

    def compute(bufs, parts):
        u_rows, e_rows, f_rows = bufs
        p0 = e_rows[0, pl.ds(0, 16)]
        p1 = e_rows[0, pl.ds(16, 16)]
        p2 = e_rows[0, pl.ds(32, 16)]
        p3 = e_rows[0, pl.ds(48, 16)]

        def row(r, c3):
            u0 = u_rows[r, pl.ds(0, 16)]
            u1 = u_rows[r, pl.ds(16, 16)]
            u2 = u_rows[r, pl.ds(32, 16)]
            u3 = u_rows[r, pl.ds(48, 16)]
            e0 = e_rows[r, pl.ds(0, 16)]
            e1 = e_rows[r, pl.ds(16, 16)]
            e2 = e_rows[r, pl.ds(32, 16)]
            e3 = e_rows[r, pl.ds(48, 16)]
            f0 = f_rows[r, pl.ds(0, 16)]
            f1 = f_rows[r, pl.ds(16, 16)]
            f2 = f_rows[r, pl.ds(32, 16)]
            f3 = f_rows[r, pl.ds(48, 16)]
            sl = pl.ds(r * 16, 16)
            parts[0][sl] = u0 * u0 + u1 * u1 + u2 * u2 + u3 * u3
            parts[1][sl] = e0 * e0 + e1 * e1 + e2 * e2 + e3 * e3
            parts[2][sl] = f0 * f0 + f1 * f1 + f2 * f2 + f3 * f3
            parts[3][sl] = f0 * p0 + f1 * p1 + f2 * p2 + f3 * p3
            parts[4][sl] = u0 * e0 + u1 * e1 + u2 * e2 + u3 * e3
            parts[5][sl] = u0 * f0 + u1 * f1 + u2 * f2 + u3 * f3
            return c3

        lax.fori_loop(0, _CH, row, 0, unroll=4)

    issue_gathers(0, bufs0, sg0)
    issue_gathers(1, bufs1, sg1)

    def step(ii, carry):
        ci0 = 2 * ii
        ci1 = 2 * ii + 1

        @pl.when(ii > 0)
        def _():
            wait_parts(parts0, so0)
        wait_gathers(bufs0, sg0)
        compute(bufs0, parts0)
        issue_parts(ci0, parts0, so0)

        @pl.when(ii < half - 1)
        def _():
            issue_gathers(ci0 + 2, bufs0, sg0)

        @pl.when(ii > 0)
        def _():
            wait_parts(parts1, so1)
        wait_gathers(bufs1, sg1)
        compute(bufs1, parts1)
        issue_parts(ci1, parts1, so1)

        @pl.when(ii < half - 1)
        def _():
            issue_gathers(ci1 + 2, bufs1, sg1)
        return carry

    lax.fori_loop(0, half, step, 0)
    wait_parts(parts0, so0)
    wait_parts(parts1, so1)


def _sc_dots(id_embedding, feature, uidx, iidx, fidx):
    mesh = plsc.VectorSubcoreMesh(core_axis_name="c", subcore_axis_name="s")
    vec = jax.ShapeDtypeStruct((N * 16,), jnp.float32)
    ivec = pltpu.VMEM((_PW,), jnp.int32)
    rows_ue = pltpu.VMEM((_CH, DIM_E), jnp.float32)
    rows_f = pltpu.VMEM((_CH, DIM_E), jnp.float32)
    part = pltpu.VMEM((_CH * 16,), jnp.float32)
    return pl.kernel(
        _sc_body,
        out_type=(vec, vec, vec, vec, vec, vec),
        mesh=mesh,
        scratch_types=[
            ivec, ivec, ivec,
            rows_ue, rows_ue, rows_f, rows_ue, rows_ue, rows_f,
            part, part, part, part, part, part,
            part, part, part, part, part, part,
            pltpu.SemaphoreType.DMA, pltpu.SemaphoreType.DMA,
            pltpu.SemaphoreType.DMA, pltpu.SemaphoreType.DMA,
        ],
        compiler_params=pltpu.CompilerParams(use_tc_tiling_on_sc=False,
                                             needs_layout_passes=False),
    )(id_embedding, feature, uidx, iidx, fidx)


_FGRID = 8
_FQ = (N // 8) // _FGRID
_FG = 128


def _fin_body(uu_ref, ee_ref, ff_ref, fp_ref, ue_ref, uf_ref, mp_ref,
              s1_ref, s2_ref, ru_ref, re_ref):
    step = pl.program_id(0)
    li = lax.broadcasted_iota(jnp.int32, (128, 8), 0) // 16
    ki = lax.broadcasted_iota(jnp.int32, (128, 8), 1)
    sel = (li == ki).astype(jnp.float32)
    red = lambda ref: lax.dot_general(ref[...], sel, (((1,), (0,)), ((), ())),
                                      preferred_element_type=jnp.float32)
    uu = red(uu_ref)
    ee = red(ee_ref)
    ff = red(ff_ref)
    fp = red(fp_ref)
    ue = red(ue_ref)
    uf = red(uf_ref)
    mp = mp_ref[...]
    um = ue + mp * (uf - ue)

    qi = lax.broadcasted_iota(jnp.int32, (_FG, _FQ), 1) // 16
    gi = lax.broadcasted_iota(jnp.int32, (_FG, _FQ), 0)
    gsel = (qi == gi).astype(jnp.float32)
    gsum = lambda x: jnp.sum(lax.dot_general(
        gsel, x, (((1,), (0,)), ((), ())),
        preferred_element_type=jnp.float32), axis=1)

    q0 = (lax.broadcasted_iota(jnp.int32, (_FQ, 8), 0) % 16 == 0)
    k0 = (lax.broadcasted_iota(jnp.int32, (_FQ, 8), 1) == 0)
    pm = (q0 & k0).astype(jnp.float32)

    pp = gsum(ee * pm)
    ppr = lax.dot_general(gsel, pp.reshape(_FG, 1), (((0,), (0,)), ((), ())),
                          preferred_element_type=jnp.float32)

    nf = jnp.maximum(jnp.sqrt(ff), 1e-12)
    npp = jnp.maximum(jnp.sqrt(ppr), 1e-12)
    d1 = fp / (nf * npp)
    e1 = jnp.exp(d1 * (1.0 / TEMP_VALUE))
    e2 = jnp.exp(um * (1.0 / TEMP_VALUE))

    tot1 = gsum(e1)
    tot2 = gsum(e2)
    pos1 = gsum(e1 * pm)
    pos2 = gsum(e2 * pm)
    s1_ref[step, :] = -jnp.log(pos1 / tot1) * CONTRASTIVE
    s2_ref[step, :] = -jnp.log(pos2 / tot2) * (1.0 - CONTRASTIVE)

    su = jnp.sum(jnp.sqrt(uu)).reshape(1, 1)
    se = jnp.sum(jnp.sqrt(ee)).reshape(1, 1)

    @pl.when(step == 0)
    def _():
        ru_ref[...] = jnp.zeros_like(ru_ref)
        re_ref[...] = jnp.zeros_like(re_ref)

    ru_ref[...] += su
    re_ref[...] += se


def _finalize(uu, ee, ff, fp, ue, uf, maskp):
    spec = pl.BlockSpec((_FQ, 128), lambda i: (i, 0))
    mspec = pl.BlockSpec((_FQ, 8), lambda i: (i, 0))
    return pl.pallas_call(
        _fin_body,
        grid=(_FGRID,),
        in_specs=[spec, spec, spec, spec, spec, spec, mspec],
        out_specs=[
            pl.BlockSpec((_FGRID, _FG), lambda i: (0, 0)),
            pl.BlockSpec((_FGRID, _FG), lambda i: (0, 0)),
            pl.BlockSpec((1, 1), lambda i: (0, 0)),
            pl.BlockSpec((1, 1), lambda i: (0, 0)),
        ],
        out_shape=(
            jax.ShapeDtypeStruct((_FGRID, _FG), jnp.float32),
            jax.ShapeDtypeStruct((_FGRID, _FG), jnp.float32),
            jax.ShapeDtypeStruct((1, 1), jnp.float32),
            jax.ShapeDtypeStruct((1, 1), jnp.float32),
        ),
    )(uu, ee, ff, fp, ue, uf, maskp)


def kernel(user_tensor, item_tensor, id_embedding, v_feat, W1, b1, W2, b2):
    feature = _mlp(v_feat, W1, b1, W2, b2)

    uidx = user_tensor.reshape(-1).astype(jnp.int32)
    iidx = item_tensor.reshape(-1).astype(jnp.int32)
    fidx = iidx - NUM_USER

    uu, ee, ff, fp, ue, uf = _sc_dots(id_embedding, feature, uidx, iidx, fidx)

    maskp = jnp.asarray(_MASK_NP.reshape(N // 8, 8))
    pk = (N // 8, 128)
    s1m, s2m, ru, re = _finalize(uu.reshape(pk), ee.reshape(pk), ff.reshape(pk),
                                 fp.reshape(pk), ue.reshape(pk), uf.reshape(pk),
                                 maskp)
    reg = (ru[0, 0] + re[0, 0]) / (2.0 * N)
    return (s1m.reshape(-1), s2m.reshape(-1), reg)
